# baseline (speedup 1.0000x reference)
import functools

import jax
import jax.numpy as jnp
from jax import lax
from jax.experimental import pallas as pl
from jax.experimental.pallas import tpu as pltpu
from jax.experimental.pallas import tpu_sc as plsc

VOCAB = 100000
EMBED = 32
BATCH = 1024

_NC = 2
_NS = 16
_NW = _NC * _NS
_B_PER_W = BATCH // _NW


def _make_sc_gather():
  mesh = plsc.VectorSubcoreMesh(
      core_axis_name="c", subcore_axis_name="s",
      num_cores=_NC, num_subcores=_NS)

  @functools.partial(
      pl.kernel,
      mesh=mesh,
      compiler_params=pltpu.CompilerParams(use_tc_tiling_on_sc=False),
      out_type=jax.ShapeDtypeStruct((BATCH, EMBED), jnp.float32),
      scratch_types=[
          pltpu.VMEM((_B_PER_W,), jnp.int32),
          pltpu.VMEM((_B_PER_W, EMBED), jnp.float32),
          pltpu.SemaphoreType.DMA,
      ],
  )
  def gather(table_hbm, idx_hbm, out_hbm, idx_v, rows_v, sem):
    wid = lax.axis_index("s") * _NC + lax.axis_index("c")
    base = wid * _B_PER_W
    pltpu.sync_copy(idx_hbm.at[pl.ds(base, _B_PER_W)], idx_v)
    pltpu.async_copy(table_hbm.at[idx_v], rows_v, sem).wait()
    pltpu.sync_copy(rows_v, out_hbm.at[pl.ds(base, _B_PER_W)])

  return gather


_sc_gather = _make_sc_gather()

_NB = 2048
_NTF = VOCAB // _NB
_LAST = VOCAB - _NTF * _NB
_R = 4


def _tile_copy(acc, o_hbm, sems, j, slot):
  return pltpu.make_async_copy(
      acc.at[slot],
      o_hbm.at[:, pl.ds(j * _NB, _NB)],
      sems.at[slot])


def _tail_copy(tacc, o_hbm, tsem):
  return pltpu.make_async_copy(
      tacc, o_hbm.at[:, pl.ds(_NTF * _NB, _LAST)], tsem)


def _dense_body(emb_ref, w_ref, b_ref, wt_ref, bt_ref, o_hbm,
                acc, sems, tacc, tsem):
  i = pl.program_id(0)
  slot = lax.rem(i, _R)

  @pl.when(i >= _R)
  def _():
    _tile_copy(acc, o_hbm, sems, i - _R, slot).wait()

  @pl.when(i < _NTF)
  def _():
    acc[slot] = lax.dot_general(
        emb_ref[...], w_ref[...],
        (((1,), (1,)), ((), ())),
        preferred_element_type=jnp.float32,
    ) + b_ref[0]
    _tile_copy(acc, o_hbm, sems, i, slot).start()

  @pl.when(i == _NTF)
  def _():
    tacc[...] = lax.dot_general(
        emb_ref[...], wt_ref[...],
        (((1,), (1,)), ((), ())),
        preferred_element_type=jnp.float32,
    ) + bt_ref[0]
    _tail_copy(tacc, o_hbm, tsem).start()
    for j in range(_NTF - _R + 1, _NTF):
      _tile_copy(acc, o_hbm, sems, j, j % _R).wait()
    _tail_copy(tacc, o_hbm, tsem).wait()


def _dense(emb, W48, b48, W_tail, b_tail):
  return pl.pallas_call(
      _dense_body,
      grid=(_NTF + 1,),
      in_specs=[
          pl.BlockSpec((BATCH, EMBED), lambda i: (0, 0)),
          pl.BlockSpec((_NB, EMBED), lambda i: (jnp.minimum(i, _NTF - 1), 0)),
          pl.BlockSpec((1, 1, _NB), lambda i: (jnp.minimum(i, _NTF - 1), 0, 0)),
          pl.BlockSpec((_LAST, EMBED), lambda i: (0, 0)),
          pl.BlockSpec((1, _LAST), lambda i: (0, 0)),
      ],
      out_specs=pl.BlockSpec(memory_space=pl.ANY),
      out_shape=jax.ShapeDtypeStruct((BATCH, VOCAB), jnp.float32),
      scratch_shapes=[
          pltpu.VMEM((_R, BATCH, _NB), jnp.float32),
          pltpu.SemaphoreType.DMA((_R,)),
          pltpu.VMEM((BATCH, _LAST), jnp.float32),
          pltpu.SemaphoreType.DMA,
      ],
  )(emb, W48, b48, W_tail, b_tail)


def kernel(x, emb_table, W, b):
  emb = _sc_gather(emb_table, x.astype(jnp.int32))
  W48 = W
  b48 = lax.slice(b, (0,), (_NTF * _NB,)).reshape(_NTF, 1, _NB)
  W_tail = lax.slice(W, (_NTF * _NB, 0), (VOCAB, EMBED))
  b_tail = lax.slice(b, (_NTF * _NB,), (VOCAB,)).reshape(1, _LAST)
  return _dense(emb, W48, b48, W_tail, b_tail)


# speedup vs baseline: 1.0016x; 1.0016x over previous
import functools

import jax
import jax.numpy as jnp
from jax import lax
from jax.experimental import pallas as pl
from jax.experimental.pallas import tpu as pltpu
from jax.experimental.pallas import tpu_sc as plsc

VOCAB = 100000
EMBED = 32
BATCH = 1024

_NC = 2
_NS = 16
_NW = _NC * _NS
_B_PER_W = BATCH // _NW


def _make_sc_gather():
  mesh = plsc.VectorSubcoreMesh(
      core_axis_name="c", subcore_axis_name="s",
      num_cores=_NC, num_subcores=_NS)

  @functools.partial(
      pl.kernel,
      mesh=mesh,
      compiler_params=pltpu.CompilerParams(use_tc_tiling_on_sc=False),
      out_type=jax.ShapeDtypeStruct((BATCH, EMBED), jnp.float32),
      scratch_types=[
          pltpu.VMEM((_B_PER_W,), jnp.int32),
          pltpu.VMEM((_B_PER_W, EMBED), jnp.float32),
          pltpu.SemaphoreType.DMA,
      ],
  )
  def gather(table_hbm, idx_hbm, out_hbm, idx_v, rows_v, sem):
    wid = lax.axis_index("s") * _NC + lax.axis_index("c")
    base = wid * _B_PER_W
    pltpu.sync_copy(idx_hbm.at[pl.ds(base, _B_PER_W)], idx_v)
    pltpu.async_copy(table_hbm.at[idx_v], rows_v, sem).wait()
    pltpu.sync_copy(rows_v, out_hbm.at[pl.ds(base, _B_PER_W)])

  return gather


_sc_gather = _make_sc_gather()

_BT = 128
_NT = 25088
_GJ = -(-VOCAB // _NT)
_GI = BATCH // _BT


def _dense_body(emb_ref, w_ref, b_ref, o_ref):
  o_ref[...] = lax.dot_general(
      emb_ref[...], w_ref[...],
      (((1,), (1,)), ((), ())),
      preferred_element_type=jnp.float32,
  ) + b_ref[...]


def _dense(emb, W, b2d):
  return pl.pallas_call(
      _dense_body,
      grid=(_GJ, _GI),
      in_specs=[
          pl.BlockSpec((_BT, EMBED), lambda j, i: (i, 0)),
          pl.BlockSpec((_NT, EMBED), lambda j, i: (j, 0)),
          pl.BlockSpec((1, _NT), lambda j, i: (0, j)),
      ],
      out_specs=pl.BlockSpec((_BT, _NT), lambda j, i: (i, j)),
      out_shape=jax.ShapeDtypeStruct((BATCH, VOCAB), jnp.float32),
  )(emb, W, b2d)


def kernel(x, emb_table, W, b):
  emb = _sc_gather(emb_table, x.astype(jnp.int32))
  return _dense(emb, W, b.reshape(1, VOCAB))


# speedup vs baseline: 1.0421x; 1.0405x over previous
import functools

import jax
import jax.numpy as jnp
from jax import lax
from jax.experimental import pallas as pl
from jax.experimental.pallas import tpu as pltpu
from jax.experimental.pallas import tpu_sc as plsc

VOCAB = 100000
EMBED = 32
BATCH = 1024

_NC = 2
_NS = 16
_NW = _NC * _NS
_B_PER_W = BATCH // _NW


def _make_sc_gather():
  mesh = plsc.VectorSubcoreMesh(
      core_axis_name="c", subcore_axis_name="s",
      num_cores=_NC, num_subcores=_NS)

  @functools.partial(
      pl.kernel,
      mesh=mesh,
      compiler_params=pltpu.CompilerParams(use_tc_tiling_on_sc=False),
      out_type=jax.ShapeDtypeStruct((BATCH, EMBED), jnp.float32),
      scratch_types=[
          pltpu.VMEM((_B_PER_W,), jnp.int32),
          pltpu.VMEM((_B_PER_W, EMBED), jnp.float32),
          pltpu.SemaphoreType.DMA,
      ],
  )
  def gather(table_hbm, idx_hbm, out_hbm, idx_v, rows_v, sem):
    wid = lax.axis_index("s") * _NC + lax.axis_index("c")
    base = wid * _B_PER_W
    pltpu.sync_copy(idx_hbm.at[pl.ds(base, _B_PER_W)], idx_v)
    pltpu.async_copy(table_hbm.at[idx_v], rows_v, sem).wait()
    pltpu.sync_copy(rows_v, out_hbm.at[pl.ds(base, _B_PER_W)])

  return gather


_sc_gather = _make_sc_gather()

_BT = 128
_NT = 25088
_GJ = -(-VOCAB // _NT)
_GI = BATCH // _BT


def _dense_body(emb_ref, w_ref, b_ref, o_ref):
  o_ref[...] = lax.dot_general(
      emb_ref[...], w_ref[...],
      (((1,), (1,)), ((), ())),
      preferred_element_type=jnp.float32,
  ) + b_ref[...]


def _dense(emb, W, b2d):
  return pl.pallas_call(
      _dense_body,
      grid=(_GJ, _GI),
      in_specs=[
          pl.BlockSpec((_BT, EMBED), lambda j, i: (i, 0)),
          pl.BlockSpec((_NT, EMBED), lambda j, i: (j, 0)),
          pl.BlockSpec((1, _NT), lambda j, i: (0, j)),
      ],
      out_specs=pl.BlockSpec((_BT, _NT), lambda j, i: (i, j)),
      out_shape=jax.ShapeDtypeStruct((BATCH, VOCAB), jnp.float32),
  )(emb, W, b2d)


def kernel(x, emb_table, W, b):
  emb = jnp.take(emb_table, x, axis=0)
  return _dense(emb, W, b.reshape(1, VOCAB))


# speedup vs baseline: 2.7057x; 2.5963x over previous
import functools

import jax
import jax.numpy as jnp
from jax import lax
from jax.experimental import pallas as pl
from jax.experimental.pallas import tpu as pltpu
from jax.experimental.pallas import tpu_sc as plsc

VOCAB = 100000
EMBED = 32
BATCH = 1024

_NC = 2
_NS = 16
_NW = _NC * _NS
_B_PER_W = BATCH // _NW


def _make_sc_gather():
  mesh = plsc.VectorSubcoreMesh(
      core_axis_name="c", subcore_axis_name="s",
      num_cores=_NC, num_subcores=_NS)

  @functools.partial(
      pl.kernel,
      mesh=mesh,
      compiler_params=pltpu.CompilerParams(use_tc_tiling_on_sc=False),
      out_type=jax.ShapeDtypeStruct((BATCH, EMBED), jnp.float32),
      scratch_types=[
          pltpu.VMEM((_B_PER_W,), jnp.int32),
          pltpu.VMEM((_B_PER_W, EMBED), jnp.float32),
          pltpu.SemaphoreType.DMA,
      ],
  )
  def gather(table_hbm, idx_hbm, out_hbm, idx_v, rows_v, sem):
    wid = lax.axis_index("s") * _NC + lax.axis_index("c")
    base = wid * _B_PER_W
    pltpu.sync_copy(idx_hbm.at[pl.ds(base, _B_PER_W)], idx_v)
    pltpu.async_copy(table_hbm.at[idx_v], rows_v, sem).wait()
    pltpu.sync_copy(rows_v, out_hbm.at[pl.ds(base, _B_PER_W)])

  return gather


_sc_gather = _make_sc_gather()

_KA = EMBED + 1
_NT = 5000
_GJ = VOCAB // _NT


def _dense_body(ea_ref, wa_ref, o_ref):
  o_ref[...] = lax.dot_general(
      wa_ref[0], ea_ref[...],
      (((0,), (1,)), ((), ())),
      preferred_element_type=jnp.float32,
  )


def _dense(ea, wa3):
  return pl.pallas_call(
      _dense_body,
      grid=(_GJ,),
      in_specs=[
          pl.BlockSpec((BATCH, _KA), lambda j: (0, 0)),
          pl.BlockSpec((1, _KA, _NT), lambda j: (j, 0, 0)),
      ],
      out_specs=pl.BlockSpec((_NT, BATCH), lambda j: (j, 0)),
      out_shape=jax.ShapeDtypeStruct((VOCAB, BATCH), jnp.float32),
  )(ea, wa3)


def kernel(x, emb_table, W, b):
  emb = _sc_gather(emb_table, x.astype(jnp.int32))
  ea = jnp.concatenate([emb, jnp.ones((BATCH, 1), jnp.float32)], axis=1)
  wa = jnp.concatenate([W.T, b[None, :]], axis=0)
  wa3 = wa.reshape(_KA, _GJ, _NT).transpose(1, 0, 2)
  return _dense(ea, wa3).T


# speedup vs baseline: 2.7078x; 1.0008x over previous
import functools

import jax
import jax.numpy as jnp
from jax import lax
from jax.experimental import pallas as pl
from jax.experimental.pallas import tpu as pltpu
from jax.experimental.pallas import tpu_sc as plsc

VOCAB = 100000
EMBED = 32
BATCH = 1024

_NC = 2
_NS = 16
_NW = _NC * _NS
_B_PER_W = BATCH // _NW


def _make_sc_gather():
  mesh = plsc.VectorSubcoreMesh(
      core_axis_name="c", subcore_axis_name="s",
      num_cores=_NC, num_subcores=_NS)

  @functools.partial(
      pl.kernel,
      mesh=mesh,
      compiler_params=pltpu.CompilerParams(use_tc_tiling_on_sc=False),
      out_type=jax.ShapeDtypeStruct((BATCH, EMBED), jnp.float32),
      scratch_types=[
          pltpu.VMEM((_B_PER_W,), jnp.int32),
          pltpu.VMEM((_B_PER_W, EMBED), jnp.float32),
          pltpu.SemaphoreType.DMA,
      ],
  )
  def gather(table_hbm, idx_hbm, out_hbm, idx_v, rows_v, sem):
    wid = lax.axis_index("s") * _NC + lax.axis_index("c")
    base = wid * _B_PER_W
    pltpu.sync_copy(idx_hbm.at[pl.ds(base, _B_PER_W)], idx_v)
    pltpu.async_copy(table_hbm.at[idx_v], rows_v, sem).wait()
    pltpu.sync_copy(rows_v, out_hbm.at[pl.ds(base, _B_PER_W)])

  return gather


_sc_gather = _make_sc_gather()

_KA = EMBED + 1
_NT = 5000
_GJ = VOCAB // _NT


def _dense_body(ea_ref, wa_ref, o_ref):
  o_ref[...] = lax.dot_general(
      wa_ref[0], ea_ref[...],
      (((0,), (1,)), ((), ())),
      preferred_element_type=jnp.float32,
  )


def _dense(ea, wa3):
  return pl.pallas_call(
      _dense_body,
      grid=(_GJ,),
      in_specs=[
          pl.BlockSpec((BATCH, _KA), lambda j: (0, 0)),
          pl.BlockSpec((1, _KA, _NT), lambda j: (j, 0, 0)),
      ],
      out_specs=pl.BlockSpec((_NT, BATCH), lambda j: (j, 0)),
      out_shape=jax.ShapeDtypeStruct((VOCAB, BATCH), jnp.float32),
  )(ea, wa3)


def kernel(x, emb_table, W, b):
  emb = _sc_gather(emb_table, x.astype(jnp.int32))
  ea = jnp.concatenate([emb, jnp.ones((BATCH, 1), jnp.float32)], axis=1)
  wt = W.reshape(_GJ, _NT, EMBED).transpose(0, 2, 1)
  wa3 = jnp.concatenate([wt, b.reshape(_GJ, 1, _NT)], axis=1)
  return _dense(ea, wa3).T
